# baseline (device time: 36949 ns/iter reference)
import functools

import jax
import jax.numpy as jnp
from jax import lax
from jax.experimental import pallas as pl
from jax.experimental.pallas import tpu as pltpu

N_Z = 4


def kernel(x):
    m_per, n = x.shape
    half = m_per // 2
    quart = m_per // 4

    def body(x_ref, out_ref, send_z, recv_z, sx, rx, sy, ry, sd, rd):
        my_x = lax.axis_index("x")
        my_y = lax.axis_index("y")
        my_z = lax.axis_index("z")

        def rows(o, xx, yy):
            return o * m_per + xx * half + yy * quart

        def copy(row0, ssem, s_slot, rsem, r_slot, dev):
            return pltpu.make_async_remote_copy(
                src_ref=out_ref.at[pl.ds(row0, quart), :],
                dst_ref=out_ref.at[pl.ds(row0, quart), :],
                send_sem=ssem.at[s_slot],
                recv_sem=rsem.at[r_slot],
                device_id=dev,
                device_id_type=pl.DeviceIdType.MESH,
            )

        def z_send(t):
            return copy(rows(my_z, my_x, my_y), send_z, t, recv_z, my_z,
                        (my_x, my_y, t))

        def z_recv(o):
            return copy(rows(o, my_x, my_y), send_z, o, recv_z, o,
                        (my_x, my_y, o))

        def x_own(o):
            return copy(rows(o, my_x, my_y), sx, o, rx, o,
                        (1 - my_x, my_y, my_z))

        def y_own(o):
            return copy(rows(o, my_x, my_y), sy, o, ry, o,
                        (my_x, 1 - my_y, my_z))

        def d_own(o):
            return copy(rows(o, my_x, my_y), sd, o, rd, o,
                        (1 - my_x, 1 - my_y, my_z))

        def xy_recv(o, xx, yy, ssem, rsem, dev):
            return copy(rows(o, xx, yy), ssem, o, rsem, o, dev)

        def neighbor_signal(sem):
            for dev in ((1 - my_x, my_y, my_z), (my_x, 1 - my_y, my_z),
                        (1 - my_x, 1 - my_y, my_z)):
                pl.semaphore_signal(
                    sem, inc=1, device_id=dev,
                    device_id_type=pl.DeviceIdType.MESH,
                )
            for t in range(N_Z):
                @pl.when(t != my_z)
                def _():
                    pl.semaphore_signal(
                        sem, inc=1, device_id=(my_x, my_y, t),
                        device_id_type=pl.DeviceIdType.MESH,
                    )

        barrier_sem = pltpu.get_barrier_semaphore()
        neighbor_signal(barrier_sem)
        pl.semaphore_wait(barrier_sem, 6)

        out_ref[pl.ds(my_z * m_per, m_per), :] = x_ref[:, :]

        for s in range(1, N_Z):
            for t in (my_z + s, my_z - s):
                @pl.when(jnp.logical_and(t >= 0, t <= N_Z - 1))
                def _():
                    z_send(t).start()

        for s in range(1, N_Z):
            for o in (my_z - s, my_z + s):
                @pl.when(jnp.logical_and(o >= 0, o <= N_Z - 1))
                def _():
                    z_recv(o).wait_recv()
                    x_own(o).start()
                    y_own(o).start()
                    d_own(o).start()

        for s in range(1, N_Z):
            for o in (my_z - s, my_z + s):
                @pl.when(jnp.logical_and(o >= 0, o <= N_Z - 1))
                def _():
                    xy_recv(o, 1 - my_x, my_y, sx, rx,
                            (1 - my_x, my_y, my_z)).wait_recv()
                    xy_recv(o, my_x, 1 - my_y, sy, ry,
                            (my_x, 1 - my_y, my_z)).wait_recv()
                    xy_recv(o, 1 - my_x, 1 - my_y, sd, rd,
                            (1 - my_x, 1 - my_y, my_z)).wait_recv()

        for t in range(N_Z):
            @pl.when(t != my_z)
            def _():
                z_send(t).wait_send()

        for o in range(N_Z):
            @pl.when(o != my_z)
            def _():
                x_own(o).wait_send()
                y_own(o).wait_send()
                d_own(o).wait_send()

        @functools.partial(
            pl.run_scoped, second_barrier=pltpu.SemaphoreType.REGULAR
        )
        def _(second_barrier):
            neighbor_signal(second_barrier)
            pl.semaphore_wait(second_barrier, 6)

    dma = pltpu.SemaphoreType.DMA((N_Z,))
    return pl.pallas_call(
        body,
        out_shape=jax.ShapeDtypeStruct((N_Z * m_per, n), x.dtype),
        in_specs=[pl.BlockSpec(memory_space=pltpu.VMEM)],
        out_specs=pl.BlockSpec(memory_space=pltpu.VMEM),
        scratch_shapes=[dma] * 8,
        compiler_params=pltpu.CompilerParams(collective_id=0),
    )(x)


# device time: 33057 ns/iter; 1.1177x vs baseline; 1.1177x over previous
import functools

import jax
import jax.numpy as jnp
from jax import lax
from jax.experimental import pallas as pl
from jax.experimental.pallas import tpu as pltpu

N_Z = 4


def kernel(x):
    m_per, n = x.shape
    half = m_per // 2
    quart = m_per // 4
    piece = m_per // 8

    def body(x_ref, out_ref,
             send_up, recv_up, send_dn, recv_dn,
             sx_own, rx_own, sy_own, ry_own,
             s_rel_x, r_diag_x, s_rel_y, r_diag_y,
             sd_dir, rd_dir):
        my_x = lax.axis_index("x")
        my_y = lax.axis_index("y")
        my_z = lax.axis_index("z")
        has_up = my_z < N_Z - 1
        has_dn = my_z > 0

        def qbase(o, xx, yy):
            return o * m_per + xx * half + yy * quart

        def copy(row0, nrows, ssem, rsem, o, dev):
            return pltpu.make_async_remote_copy(
                src_ref=out_ref.at[pl.ds(row0, nrows), :],
                dst_ref=out_ref.at[pl.ds(row0, nrows), :],
                send_sem=ssem.at[o],
                recv_sem=rsem.at[o],
                device_id=dev,
                device_id_type=pl.DeviceIdType.MESH,
            )

        def z_copy(o, dz, ssem, rsem):
            return copy(qbase(o, my_x, my_y), quart, ssem, rsem, o,
                        (my_x, my_y, my_z + dz))

        def x_own(o):
            return copy(qbase(o, my_x, my_y), quart, sx_own, rx_own, o,
                        (1 - my_x, my_y, my_z))

        def y_own(o):
            return copy(qbase(o, my_x, my_y), quart, sy_own, ry_own, o,
                        (my_x, 1 - my_y, my_z))

        def d_dir(o):
            return copy(qbase(o, my_x, my_y), quart, sd_dir, rd_dir, o,
                        (1 - my_x, 1 - my_y, my_z))

        def rel_x(o):
            return copy(qbase(o, my_x, 1 - my_y), piece, s_rel_x, r_diag_x,
                        o, (1 - my_x, my_y, my_z))

        def rel_y(o):
            return copy(qbase(o, 1 - my_x, my_y) + piece, piece, s_rel_y,
                        r_diag_y, o, (my_x, 1 - my_y, my_z))

        def diag_x_desc(o):
            return copy(qbase(o, 1 - my_x, 1 - my_y), piece, s_rel_x,
                        r_diag_x, o, (1 - my_x, my_y, my_z))

        def diag_y_desc(o):
            return copy(qbase(o, 1 - my_x, 1 - my_y) + piece, piece,
                        s_rel_y, r_diag_y, o, (my_x, 1 - my_y, my_z))

        def diag_dir_desc(o):
            return copy(qbase(o, 1 - my_x, 1 - my_y), quart, sd_dir,
                        rd_dir, o, (1 - my_x, 1 - my_y, my_z))

        def xy_own_recv(o, xx, yy, ssem, rsem, dev):
            return copy(qbase(o, xx, yy), quart, ssem, rsem, o, dev)

        def neighbor_signal(sem):
            for dev in ((1 - my_x, my_y, my_z), (my_x, 1 - my_y, my_z),
                        (1 - my_x, 1 - my_y, my_z)):
                pl.semaphore_signal(
                    sem, inc=1, device_id=dev,
                    device_id_type=pl.DeviceIdType.MESH,
                )
            @pl.when(has_up)
            def _():
                pl.semaphore_signal(
                    sem, inc=1, device_id=(my_x, my_y, my_z + 1),
                    device_id_type=pl.DeviceIdType.MESH,
                )
            @pl.when(has_dn)
            def _():
                pl.semaphore_signal(
                    sem, inc=1, device_id=(my_x, my_y, my_z - 1),
                    device_id_type=pl.DeviceIdType.MESH,
                )

        def neighbor_wait(sem):
            is_middle = jnp.logical_and(has_up, has_dn)
            @pl.when(is_middle)
            def _():
                pl.semaphore_wait(sem, 5)
            @pl.when(jnp.logical_not(is_middle))
            def _():
                pl.semaphore_wait(sem, 4)

        out_ref[pl.ds(my_z * m_per, m_per), :] = x_ref[:, :]

        barrier_sem = pltpu.get_barrier_semaphore()
        neighbor_signal(barrier_sem)
        neighbor_wait(barrier_sem)

        for s in range(1, N_Z):
            o_us = my_z - s + 1
            o_ds = my_z + s - 1
            o_ur = my_z - s
            o_dr = my_z + s

            @pl.when(jnp.logical_and(has_up, o_us >= 0))
            def _():
                z_copy(o_us, 1, send_up, recv_up).start()

            @pl.when(jnp.logical_and(has_dn, o_ds <= N_Z - 1))
            def _():
                z_copy(o_ds, -1, send_dn, recv_dn).start()

            @pl.when(o_ur >= 0)
            def _():
                z_copy(o_ur, 1, send_up, recv_up).wait_recv()
                x_own(o_ur).start()
                y_own(o_ur).start()
                if s == N_Z - 1:
                    d_dir(o_ur).start()

            @pl.when(o_dr <= N_Z - 1)
            def _():
                z_copy(o_dr, -1, send_dn, recv_dn).wait_recv()
                x_own(o_dr).start()
                y_own(o_dr).start()
                if s == N_Z - 1:
                    d_dir(o_dr).start()

        for s in range(1, N_Z - 1):
            for o in (my_z - s, my_z + s):
                @pl.when(jnp.logical_and(o >= 0, o <= N_Z - 1))
                def _():
                    y_own(o).wait_recv()
                    rel_x(o).start()
                    x_own(o).wait_recv()
                    rel_y(o).start()

        for o in (my_z - (N_Z - 1), my_z + (N_Z - 1)):
            @pl.when(jnp.logical_and(o >= 0, o <= N_Z - 1))
            def _():
                y_own(o).wait_recv()
                x_own(o).wait_recv()

        for s in range(1, N_Z - 1):
            for o in (my_z - s, my_z + s):
                @pl.when(jnp.logical_and(o >= 0, o <= N_Z - 1))
                def _():
                    diag_x_desc(o).wait_recv()
                    diag_y_desc(o).wait_recv()

        for o in (my_z - (N_Z - 1), my_z + (N_Z - 1)):
            @pl.when(jnp.logical_and(o >= 0, o <= N_Z - 1))
            def _():
                diag_dir_desc(o).wait_recv()

        for s in range(1, N_Z):
            o_us = my_z - s + 1
            o_ds = my_z + s - 1

            @pl.when(jnp.logical_and(has_up, o_us >= 0))
            def _():
                z_copy(o_us, 1, send_up, recv_up).wait_send()

            @pl.when(jnp.logical_and(has_dn, o_ds <= N_Z - 1))
            def _():
                z_copy(o_ds, -1, send_dn, recv_dn).wait_send()

        for o in range(N_Z):
            @pl.when(o != my_z)
            def _():
                x_own(o).wait_send()
                y_own(o).wait_send()

        for s in range(1, N_Z - 1):
            for o in (my_z - s, my_z + s):
                @pl.when(jnp.logical_and(o >= 0, o <= N_Z - 1))
                def _():
                    rel_x(o).wait_send()
                    rel_y(o).wait_send()

        for o in (my_z - (N_Z - 1), my_z + (N_Z - 1)):
            @pl.when(jnp.logical_and(o >= 0, o <= N_Z - 1))
            def _():
                d_dir(o).wait_send()

        @functools.partial(
            pl.run_scoped, second_barrier=pltpu.SemaphoreType.REGULAR
        )
        def _(second_barrier):
            neighbor_signal(second_barrier)
            neighbor_wait(second_barrier)

    dma = pltpu.SemaphoreType.DMA((N_Z,))
    return pl.pallas_call(
        body,
        out_shape=jax.ShapeDtypeStruct((N_Z * m_per, n), x.dtype),
        in_specs=[pl.BlockSpec(memory_space=pltpu.VMEM)],
        out_specs=pl.BlockSpec(memory_space=pltpu.VMEM),
        scratch_shapes=[dma] * 14,
        compiler_params=pltpu.CompilerParams(collective_id=0),
    )(x)


# device time: 30609 ns/iter; 1.2071x vs baseline; 1.0800x over previous
import functools

import jax
import jax.numpy as jnp
from jax import lax
from jax.experimental import pallas as pl
from jax.experimental.pallas import tpu as pltpu

N_Z = 4


def kernel(x):
    m_per, n = x.shape
    half = m_per // 2
    quart = m_per // 4
    piece = m_per // 8

    def body(x_ref, out_ref,
             send_up, recv_up, send_dn, recv_dn,
             sx_own, rx_own, sy_own, ry_own,
             s_rel_x, r_diag_x, s_rel_y, r_diag_y):
        my_x = lax.axis_index("x")
        my_y = lax.axis_index("y")
        my_z = lax.axis_index("z")
        has_up = my_z < N_Z - 1
        has_dn = my_z > 0

        def qbase(o, xx, yy):
            return o * m_per + xx * half + yy * quart

        def copy(row0, nrows, ssem, rsem, o, dev):
            return pltpu.make_async_remote_copy(
                src_ref=out_ref.at[pl.ds(row0, nrows), :],
                dst_ref=out_ref.at[pl.ds(row0, nrows), :],
                send_sem=ssem.at[o],
                recv_sem=rsem.at[o],
                device_id=dev,
                device_id_type=pl.DeviceIdType.MESH,
            )

        def z_copy(o, dz, ssem, rsem):
            return copy(qbase(o, my_x, my_y), quart, ssem, rsem, o,
                        (my_x, my_y, my_z + dz))

        def x_own(o):
            return copy(qbase(o, my_x, my_y), quart, sx_own, rx_own, o,
                        (1 - my_x, my_y, my_z))

        def y_own(o):
            return copy(qbase(o, my_x, my_y), quart, sy_own, ry_own, o,
                        (my_x, 1 - my_y, my_z))

        def rel_x(o):
            return copy(qbase(o, my_x, 1 - my_y), piece, s_rel_x, r_diag_x,
                        o, (1 - my_x, my_y, my_z))

        def rel_y(o):
            return copy(qbase(o, 1 - my_x, my_y) + piece, piece, s_rel_y,
                        r_diag_y, o, (my_x, 1 - my_y, my_z))

        def diag_x_desc(o):
            return copy(qbase(o, 1 - my_x, 1 - my_y), piece, s_rel_x,
                        r_diag_x, o, (1 - my_x, my_y, my_z))

        def diag_y_desc(o):
            return copy(qbase(o, 1 - my_x, 1 - my_y) + piece, piece,
                        s_rel_y, r_diag_y, o, (my_x, 1 - my_y, my_z))

        def neighbor_signal(sem):
            for dev in ((1 - my_x, my_y, my_z), (my_x, 1 - my_y, my_z)):
                pl.semaphore_signal(
                    sem, inc=1, device_id=dev,
                    device_id_type=pl.DeviceIdType.MESH,
                )
            @pl.when(has_up)
            def _():
                pl.semaphore_signal(
                    sem, inc=1, device_id=(my_x, my_y, my_z + 1),
                    device_id_type=pl.DeviceIdType.MESH,
                )
            @pl.when(has_dn)
            def _():
                pl.semaphore_signal(
                    sem, inc=1, device_id=(my_x, my_y, my_z - 1),
                    device_id_type=pl.DeviceIdType.MESH,
                )

        def neighbor_wait(sem):
            is_middle = jnp.logical_and(has_up, has_dn)
            @pl.when(is_middle)
            def _():
                pl.semaphore_wait(sem, 4)
            @pl.when(jnp.logical_not(is_middle))
            def _():
                pl.semaphore_wait(sem, 3)

        barrier_sem = pltpu.get_barrier_semaphore()
        neighbor_signal(barrier_sem)
        neighbor_wait(barrier_sem)

        out_ref[pl.ds(my_z * m_per, m_per), :] = x_ref[:, :]

        for s in range(1, N_Z):
            o_us = my_z - s + 1
            o_ds = my_z + s - 1
            o_ur = my_z - s
            o_dr = my_z + s

            @pl.when(jnp.logical_and(has_up, o_us >= 0))
            def _():
                z_copy(o_us, 1, send_up, recv_up).start()

            @pl.when(jnp.logical_and(has_dn, o_ds <= N_Z - 1))
            def _():
                z_copy(o_ds, -1, send_dn, recv_dn).start()

            @pl.when(o_ur >= 0)
            def _():
                z_copy(o_ur, 1, send_up, recv_up).wait_recv()
                x_own(o_ur).start()
                y_own(o_ur).start()

            @pl.when(o_dr <= N_Z - 1)
            def _():
                z_copy(o_dr, -1, send_dn, recv_dn).wait_recv()
                x_own(o_dr).start()
                y_own(o_dr).start()

        for s in range(1, N_Z):
            for o in (my_z - s, my_z + s):
                @pl.when(jnp.logical_and(o >= 0, o <= N_Z - 1))
                def _():
                    y_own(o).wait_recv()
                    rel_x(o).start()
                    x_own(o).wait_recv()
                    rel_y(o).start()

        for s in range(1, N_Z):
            for o in (my_z - s, my_z + s):
                @pl.when(jnp.logical_and(o >= 0, o <= N_Z - 1))
                def _():
                    diag_x_desc(o).wait_recv()
                    diag_y_desc(o).wait_recv()

        for s in range(1, N_Z):
            o_us = my_z - s + 1
            o_ds = my_z + s - 1

            @pl.when(jnp.logical_and(has_up, o_us >= 0))
            def _():
                z_copy(o_us, 1, send_up, recv_up).wait_send()

            @pl.when(jnp.logical_and(has_dn, o_ds <= N_Z - 1))
            def _():
                z_copy(o_ds, -1, send_dn, recv_dn).wait_send()

        for o in range(N_Z):
            @pl.when(o != my_z)
            def _():
                x_own(o).wait_send()
                y_own(o).wait_send()
                rel_x(o).wait_send()
                rel_y(o).wait_send()

        @functools.partial(
            pl.run_scoped, second_barrier=pltpu.SemaphoreType.REGULAR
        )
        def _(second_barrier):
            neighbor_signal(second_barrier)
            neighbor_wait(second_barrier)

    dma = pltpu.SemaphoreType.DMA((N_Z,))
    return pl.pallas_call(
        body,
        out_shape=jax.ShapeDtypeStruct((N_Z * m_per, n), x.dtype),
        in_specs=[pl.BlockSpec(memory_space=pltpu.VMEM)],
        out_specs=pl.BlockSpec(memory_space=pltpu.VMEM),
        scratch_shapes=[dma] * 12,
        compiler_params=pltpu.CompilerParams(collective_id=0),
    )(x)
